# baseline (speedup 1.0000x reference)
import functools

import jax
import jax.numpy as jnp
from jax import lax
from jax.experimental import pallas as pl
from jax.experimental.pallas import tpu as pltpu
from jax.experimental.pallas import tpu_sc as plsc

_LANE = 128
_E = 16
_BLK = 65536

_sc_params = pltpu.CompilerParams(needs_layout_passes=False,
                                  use_tc_tiling_on_sc=True)


def _colsum_rows(x_ref, o_ref, rows, col0=0):
    for r in range(rows):
        o_ref[pl.ds(r, 1), :] = jnp.sum(
            x_ref[:, pl.ds(col0 + r * _LANE, _LANE)], axis=0, keepdims=True)


@functools.lru_cache(maxsize=None)
def _build_colsum_tc(V1, V2):
    grid = (V1 + _BLK - 1) // _BLK
    rows_per_blk = _BLK // _LANE
    n_rows1 = grid * rows_per_blk
    blk2 = ((V2 + _LANE - 1) // _LANE) * _LANE
    n_rows2 = blk2 // _LANE

    def body(x1_ref, x2_ref, o1_ref, o2_ref):
        _colsum_rows(x1_ref, o1_ref, rows_per_blk)

        @pl.when(pl.program_id(0) == 0)
        def _():
            _colsum_rows(x2_ref, o2_ref, n_rows2)

    return pl.pallas_call(
        body,
        grid=(grid,),
        in_specs=[pl.BlockSpec((_E, _BLK), lambda j: (0, j)),
                  pl.BlockSpec((_E, blk2), lambda j: (0, 0))],
        out_specs=[pl.BlockSpec((rows_per_blk, _LANE), lambda j: (j, 0)),
                   pl.BlockSpec((n_rows2, _LANE), lambda j: (0, 0))],
        out_shape=[jax.ShapeDtypeStruct((n_rows1, _LANE), jnp.float32),
                   jax.ShapeDtypeStruct((n_rows2, _LANE), jnp.float32)],
    )


@functools.lru_cache(maxsize=None)
def _build_lookup(B, n_rows1, n_rows2):
    mesh = plsc.VectorSubcoreMesh(core_axis_name="c", subcore_axis_name="s")
    info = plsc.get_sparse_core_info()
    NC, NS, L = info.num_cores, info.num_subcores, info.num_lanes
    NW = NC * NS
    b_per_w = B // NW
    n_chunks = b_per_w // _LANE
    n_groups = b_per_w // L

    @functools.partial(
        pl.kernel,
        mesh=mesh,
        out_type=jax.ShapeDtypeStruct((B,), jnp.float32),
        compiler_params=_sc_params,
        scratch_types=[
            pltpu.VMEM((n_chunks, _LANE), jnp.int32),
            pltpu.VMEM((n_chunks, _LANE), jnp.int32),
            pltpu.VMEM((n_chunks, _LANE), jnp.int32),
            pltpu.VMEM((b_per_w, _LANE), jnp.float32),
            pltpu.VMEM((n_rows2, _LANE), jnp.float32),
            pltpu.VMEM((b_per_w,), jnp.float32),
            pltpu.SemaphoreType.DMA,
        ],
    )
    def _lookup(b1_hbm, b2_hbm, rs1_hbm, rs2_hbm, out_hbm,
                idx1_v, idx2_v, row_v, rows_v, rs2_v, out_v, sem):
        wid = lax.axis_index("s") * NC + lax.axis_index("c")
        base = wid * b_per_w

        pltpu.sync_copy(b1_hbm.at[pl.ds(wid * n_chunks, n_chunks)], idx1_v)
        pltpu.sync_copy(b2_hbm.at[pl.ds(wid * n_chunks, n_chunks)], idx2_v)
        pltpu.async_copy(rs2_hbm, rs2_v, sem)

        for j in range(n_chunks):
            for cg in range(_LANE // _E):
                sl = pl.ds(cg * _E, _E)
                row_v[j, sl] = lax.shift_right_logical(idx1_v[j, sl], 7)

        for j in range(n_chunks):
            pltpu.async_copy(rs1_hbm.at[row_v.at[j]],
                             rows_v.at[pl.ds(j * _LANE, _LANE)], sem)
        pltpu.make_async_copy(rs1_hbm.at[pl.ds(0, b_per_w)], rows_v,
                              sem).wait()
        pltpu.make_async_copy(rs2_hbm, rs2_v, sem).wait()

        lane = lax.iota(jnp.int32, L)
        for g in range(n_groups):
            sl = pl.ds((g % 8) * _E, _E)
            i1 = idx1_v[g // 8, sl]
            i2 = idx2_v[g // 8, sl]
            v1 = plsc.load_gather(rows_v, [g * L + lane, i1 & 127])
            v2 = plsc.load_gather(rs2_v,
                                  [lax.shift_right_logical(i2, 7), i2 & 127])
            out_v[pl.ds(g * L, L)] = jnp.exp(v1 + v2)

        pltpu.sync_copy(out_v, out_hbm.at[pl.ds(base, b_per_w)])

    return _lookup


def kernel(batch_1, batch_2, emb1, emb2):
    B = batch_1.shape[0]
    V1 = emb1.shape[0]
    V2 = emb2.shape[0]

    b1 = batch_1.astype(jnp.int32).reshape(-1, _LANE)
    b2 = batch_2.astype(jnp.int32).reshape(-1, _LANE)

    colsum = _build_colsum_tc(V1, V2)
    rs1, rs2 = colsum(emb1.T, emb2.T)

    lookup = _build_lookup(B, rs1.shape[0], rs2.shape[0])
    return lookup(b1, b2, rs1, rs2)


# speedup vs baseline: 1.0569x; 1.0569x over previous
import functools

import jax
import jax.numpy as jnp
from jax import lax
from jax.experimental import pallas as pl
from jax.experimental.pallas import tpu as pltpu
from jax.experimental.pallas import tpu_sc as plsc

_LANE = 128
_E = 16
_BLK = 131072

_sc_params = pltpu.CompilerParams(needs_layout_passes=False,
                                  use_tc_tiling_on_sc=True)


def _colsum_rows(x_ref, o_ref, rows, col0=0):
    for r in range(rows):
        o_ref[pl.ds(r, 1), :] = jnp.sum(
            x_ref[:, pl.ds(col0 + r * _LANE, _LANE)], axis=0, keepdims=True)


@functools.lru_cache(maxsize=None)
def _build_colsum_tc(V1, V2):
    grid = (V1 + _BLK - 1) // _BLK
    rows_per_blk = _BLK // _LANE
    n_rows1 = grid * rows_per_blk
    blk2 = ((V2 + _LANE - 1) // _LANE) * _LANE
    n_rows2 = blk2 // _LANE

    def body(x1_ref, x2_ref, o1_ref, o2_ref):
        _colsum_rows(x1_ref, o1_ref, rows_per_blk)

        @pl.when(pl.program_id(0) == 0)
        def _():
            _colsum_rows(x2_ref, o2_ref, n_rows2)

    return pl.pallas_call(
        body,
        grid=(grid,),
        in_specs=[pl.BlockSpec((_E, _BLK), lambda j: (0, j)),
                  pl.BlockSpec((_E, blk2), lambda j: (0, 0))],
        out_specs=[pl.BlockSpec((rows_per_blk, _LANE), lambda j: (j, 0)),
                   pl.BlockSpec((n_rows2, _LANE), lambda j: (0, 0))],
        out_shape=[jax.ShapeDtypeStruct((n_rows1, _LANE), jnp.float32),
                   jax.ShapeDtypeStruct((n_rows2, _LANE), jnp.float32)],
    )


@functools.lru_cache(maxsize=None)
def _build_lookup(B, n_rows1, n_rows2):
    mesh = plsc.VectorSubcoreMesh(core_axis_name="c", subcore_axis_name="s")
    info = plsc.get_sparse_core_info()
    NC, NS, L = info.num_cores, info.num_subcores, info.num_lanes
    NW = NC * NS
    b_per_w = B // NW
    n_chunks = b_per_w // _LANE
    n_groups = b_per_w // L

    @functools.partial(
        pl.kernel,
        mesh=mesh,
        out_type=jax.ShapeDtypeStruct((B,), jnp.float32),
        compiler_params=_sc_params,
        scratch_types=[
            pltpu.VMEM((n_chunks, _LANE), jnp.int32),
            pltpu.VMEM((n_chunks, _LANE), jnp.int32),
            pltpu.VMEM((n_chunks, _LANE), jnp.int32),
            pltpu.VMEM((b_per_w, _LANE), jnp.float32),
            pltpu.VMEM((n_rows2, _LANE), jnp.float32),
            pltpu.VMEM((b_per_w,), jnp.float32),
            pltpu.SemaphoreType.DMA,
        ],
    )
    def _lookup(b1_hbm, b2_hbm, rs1_hbm, rs2_hbm, out_hbm,
                idx1_v, idx2_v, row_v, rows_v, rs2_v, out_v, sem):
        wid = lax.axis_index("s") * NC + lax.axis_index("c")
        base = wid * b_per_w

        pltpu.sync_copy(b1_hbm.at[pl.ds(wid * n_chunks, n_chunks)], idx1_v)
        pltpu.sync_copy(b2_hbm.at[pl.ds(wid * n_chunks, n_chunks)], idx2_v)
        pltpu.async_copy(rs2_hbm, rs2_v, sem)

        for j in range(n_chunks):
            for cg in range(_LANE // _E):
                sl = pl.ds(cg * _E, _E)
                row_v[j, sl] = lax.shift_right_logical(idx1_v[j, sl], 7)

        for j in range(n_chunks):
            pltpu.async_copy(rs1_hbm.at[row_v.at[j]],
                             rows_v.at[pl.ds(j * _LANE, _LANE)], sem)
        pltpu.make_async_copy(rs1_hbm.at[pl.ds(0, b_per_w)], rows_v,
                              sem).wait()
        pltpu.make_async_copy(rs2_hbm, rs2_v, sem).wait()

        lane = lax.iota(jnp.int32, L)
        for g in range(n_groups):
            sl = pl.ds((g % 8) * _E, _E)
            i1 = idx1_v[g // 8, sl]
            i2 = idx2_v[g // 8, sl]
            v1 = plsc.load_gather(rows_v, [g * L + lane, i1 & 127])
            v2 = plsc.load_gather(rs2_v,
                                  [lax.shift_right_logical(i2, 7), i2 & 127])
            out_v[pl.ds(g * L, L)] = jnp.exp(v1 + v2)

        pltpu.sync_copy(out_v, out_hbm.at[pl.ds(base, b_per_w)])

    return _lookup


def kernel(batch_1, batch_2, emb1, emb2):
    B = batch_1.shape[0]
    V1 = emb1.shape[0]
    V2 = emb2.shape[0]

    b1 = batch_1.astype(jnp.int32).reshape(-1, _LANE)
    b2 = batch_2.astype(jnp.int32).reshape(-1, _LANE)

    colsum = _build_colsum_tc(V1, V2)
    rs1, rs2 = colsum(emb1.T, emb2.T)

    lookup = _build_lookup(B, rs1.shape[0], rs2.shape[0])
    return lookup(b1, b2, rs1, rs2)
